# baseline (speedup 1.0000x reference)
import functools

import jax
import jax.numpy as jnp
from jax import lax
from jax.experimental import pallas as pl
from jax.experimental.pallas import tpu as pltpu
from jax.experimental.pallas import tpu_sc as plsc

B, S, D = 1, 2048, 768
H, HKV, HD = 12, 4, 64
E, K, FF = 8, 2, 2048
EPS = 1e-5
THETA = 1e6

BS_QKV = 256
BQ = 512
BK = 1024
BS_POST = 256
BS_POS = 512
EPAD = 128

NTOK = S * K
TM = 256
NT = NTOK // TM
BF = 1024
NF = FF // BF

NSC_WORKERS = 32
CHUNK = NTOK // NSC_WORKERS

NEG = -1e30


def _dot(a, b):
    return jnp.dot(a, b, preferred_element_type=jnp.float32)


def _qkv_kernel(x_ref, ln1_ref, wq_ref, wqr_ref, wk_ref, wkr_ref, wv_ref,
                cosq_ref, sinq_ref, cosk_ref, sink_ref,
                q_ref, k_ref, v_ref):
    x = x_ref[...]
    var = jnp.mean(x * x, axis=1, keepdims=True)
    xn = (x * jax.lax.rsqrt(var + EPS) * ln1_ref[...]).astype(jnp.bfloat16)
    q = _dot(xn, wq_ref[...])
    qr = _dot(xn, wqr_ref[...])
    q_ref[...] = (q * cosq_ref[...] + qr * sinq_ref[...]).astype(jnp.bfloat16)
    k = _dot(xn, wk_ref[...])
    kr = _dot(xn, wkr_ref[...])
    k_ref[...] = (k * cosk_ref[...] + kr * sink_ref[...]).astype(jnp.bfloat16)
    v_ref[...] = _dot(xn, wv_ref[...]).astype(jnp.bfloat16)


def _attn_kernel(q_ref, k_ref, v_ref, o_ref):
    i = pl.program_id(1)
    q = q_ref[0]
    nj = (i * BQ + BQ - 1) // BK + 1

    def body(j, carry):
        m_old, l_old, acc_old = carry
        base = pl.multiple_of(j * BK, BK)
        k = k_ref[0, pl.ds(base, BK), :]
        s = _dot(q, k.T) * (1.0 / 8.0)
        rows = i * BQ + jax.lax.broadcasted_iota(jnp.int32, (BQ, BK), 0)
        cols = j * BK + jax.lax.broadcasted_iota(jnp.int32, (BQ, BK), 1)
        s = jnp.where(cols <= rows, s, NEG)
        m_new = jnp.maximum(m_old, jnp.max(s, axis=1, keepdims=True))
        alpha = jnp.exp(m_old - m_new)
        p = jnp.exp(s - m_new)
        l_new = l_old * alpha + jnp.sum(p, axis=1, keepdims=True)
        v = v_ref[0, pl.ds(base, BK), :]
        acc_new = acc_old * alpha + _dot(p.astype(jnp.bfloat16), v)
        return m_new, l_new, acc_new

    m0 = jnp.full((BQ, 1), NEG, jnp.float32)
    l0 = jnp.zeros((BQ, 1), jnp.float32)
    a0 = jnp.zeros((BQ, HD), jnp.float32)
    m, l, acc = jax.lax.fori_loop(0, nj, body, (m0, l0, a0))
    o_ref[0] = (acc / l).astype(jnp.bfloat16)


def _post_kernel(ctx_ref, res_ref, wo_ref, ln2_ref, wr_ref,
                 hid_ref, h2_ref, p_ref, oh0_ref, oh1_ref, cnt_ref,
                 cnt_acc):
    t = pl.program_id(0)
    hid = res_ref[...] + _dot(ctx_ref[...], wo_ref[...])
    hid_ref[...] = hid
    var = jnp.mean(hid * hid, axis=1, keepdims=True)
    h2 = hid * jax.lax.rsqrt(var + EPS) * ln2_ref[...]
    h2_ref[...] = h2
    logits = _dot(h2, wr_ref[...])
    cols = jax.lax.broadcasted_iota(jnp.int32, (BS_POST, EPAD), 1)
    logits = jnp.where(cols < E, logits, NEG)
    m1 = jnp.max(logits, axis=1, keepdims=True)
    i1 = jnp.min(jnp.where(logits == m1, cols, EPAD), axis=1, keepdims=True)
    masked = jnp.where(cols == i1, NEG, logits)
    m2 = jnp.max(masked, axis=1, keepdims=True)
    i2 = jnp.min(jnp.where(masked == m2, cols, EPAD), axis=1, keepdims=True)
    p1 = 1.0 / (1.0 + jnp.exp(m2 - m1))
    p2 = 1.0 - p1
    p_ref[...] = jnp.where(cols == 0, p1, 0.0) + jnp.where(cols == 1, p2, 0.0)
    oh0 = jnp.where(cols == i1, 1.0, 0.0)
    oh1 = jnp.where(cols == i2, 1.0, 0.0)
    oh0_ref[...] = oh0
    oh1_ref[...] = oh1

    @pl.when(t == 0)
    def _():
        cnt_acc[...] = jnp.zeros_like(cnt_acc)

    cnt_acc[...] += jnp.sum(oh0 + oh1, axis=0, keepdims=True)

    @pl.when(t == pl.num_programs(0) - 1)
    def _():
        cnt_ref[...] = cnt_acc[...]


def _pos_kernel(oh_ref, cnt_ref, lt_ref, ltb_ref, pos_ref, offs_ref,
                offs_acc, carry_acc):
    t = pl.program_id(0)

    @pl.when(t == 0)
    def _():
        offs_acc[...] = jnp.dot(cnt_ref[...], lt_ref[...],
                                preferred_element_type=jnp.float32,
                                precision=jax.lax.Precision.HIGHEST)
        carry_acc[...] = jnp.zeros_like(carry_acc)
        offs_ref[...] = offs_acc[...]

    oh = oh_ref[...]
    ecs = _dot(ltb_ref[...], oh)
    posmat = oh * (offs_acc[...] + carry_acc[...] + ecs)
    pos = jnp.sum(posmat, axis=1, keepdims=True)
    pos_ref[...] = pos.astype(jnp.int32)
    carry_acc[...] += jnp.sum(oh, axis=0, keepdims=True)


def _sc_scatter_body(h2_hbm, pos_hbm, hs_hbm, idx_v, rows_v, sem):
    wid = lax.axis_index("s") * 2 + lax.axis_index("c")
    base = wid * CHUNK
    tbase = base % S
    pltpu.sync_copy(pos_hbm.at[pl.ds(base, CHUNK)], idx_v)
    pltpu.sync_copy(h2_hbm.at[pl.ds(tbase, CHUNK)], rows_v)
    pltpu.async_copy(rows_v, hs_hbm.at[idx_v], sem).wait()


def _sc_gather_body(ys_hbm, pos_hbm, g2_hbm, idx_v, rows_v, sem):
    wid = lax.axis_index("s") * 2 + lax.axis_index("c")
    base = wid * CHUNK
    pltpu.sync_copy(pos_hbm.at[pl.ds(base, CHUNK)], idx_v)
    pltpu.async_copy(ys_hbm.at[idx_v], rows_v, sem).wait()
    pltpu.sync_copy(rows_v, g2_hbm.at[pl.ds(base, CHUNK)])


def _moe_kernel(offs_ref, hs_ref, wg_ref, wu_ref, wd_ref, ys_ref,
                wgs, wus, wds):
    e = pl.program_id(0)
    f = pl.program_id(1)
    t = pl.program_id(2)

    @pl.when((e == 0) & (f == 0) & (t == 0))
    def _():
        ys_ref[...] = jnp.zeros_like(ys_ref)

    start = offs_ref[e]
    end = offs_ref[e + 1]
    lo = jnp.maximum(t * TM, start)
    hi = jnp.minimum((t + 1) * TM, end)

    @pl.when((hi > lo) & (t == start // TM))
    def _():
        wgs[...] = wg_ref[0].astype(jnp.bfloat16)
        wus[...] = wu_ref[0].astype(jnp.bfloat16)
        wds[...] = wd_ref[0].astype(jnp.bfloat16)

    @pl.when(hi > lo)
    def _():
        rows = hs_ref[pl.ds(t * TM, TM), :].astype(jnp.bfloat16)
        g = jax.nn.silu(_dot(rows, wgs[...])) * _dot(rows, wus[...])
        y = _dot(g.astype(jnp.bfloat16), wds[...])
        rid = t * TM + jax.lax.broadcasted_iota(jnp.int32, (TM, 1), 0)
        mask = (rid >= lo) & (rid < hi)
        acc = ys_ref[pl.ds(t * TM, TM), :].astype(jnp.float32)
        ys_ref[pl.ds(t * TM, TM), :] = (
            acc + jnp.where(mask, y, 0.0)).astype(jnp.bfloat16)


def _combine_kernel(hid_ref, p_ref, g0_ref, g1_ref, out_ref):
    p1 = p_ref[:, 0:1]
    p2 = p_ref[:, 1:2]
    out_ref[...] = hid_ref[...] + p1 * g0_ref[0] + p2 * g1_ref[0]


def _sc_permute(src, pos, scatter):
    dt = src.dtype
    if dt != jnp.float32:
        src = lax.bitcast_convert_type(
            src.reshape(src.shape[0], -1, 2), jnp.uint32)
    w = src.shape[1]
    mesh = plsc.VectorSubcoreMesh(core_axis_name="c", subcore_axis_name="s",
                                  num_cores=2, num_subcores=16)
    body = _sc_scatter_body if scatter else _sc_gather_body
    out = pl.kernel(
        body, mesh=mesh,
        out_type=jax.ShapeDtypeStruct((NTOK, w), src.dtype),
        scratch_types=[
            pltpu.VMEM((CHUNK,), jnp.int32),
            pltpu.VMEM((CHUNK, w), src.dtype),
            pltpu.SemaphoreType.DMA,
        ],
    )(src, pos)
    if dt != jnp.float32:
        out = lax.bitcast_convert_type(out, dt).reshape(NTOK, D)
    return out


def _rot_cols(w, heads):
    w4 = w.reshape(w.shape[0], heads, 2, HD // 2)
    return jnp.concatenate([-w4[:, :, 1], w4[:, :, 0]], axis=2).reshape(w.shape)


def kernel(hidden_states, position_ids, ln1_w, ln2_w, wq, wk, wv, wo,
           w_router, wg, wu, wd):
    x = hidden_states.reshape(S, D)

    inv_freq = 1.0 / (THETA ** (jnp.arange(0, HD, 2, dtype=jnp.float32) / HD))
    pos_f = position_ids.reshape(S).astype(jnp.float32)
    freqs = pos_f[:, None] * inv_freq[None, :]
    cos1 = jnp.cos(freqs)
    sin1 = jnp.sin(freqs)
    cosh = jnp.concatenate([cos1, cos1], axis=1)
    sinh = jnp.concatenate([sin1, sin1], axis=1)
    cosq = jnp.tile(cosh, (1, H))
    sinq = jnp.tile(sinh, (1, H))
    cosk = jnp.tile(cosh, (1, HKV))
    sink = jnp.tile(sinh, (1, HKV))

    bf = jnp.bfloat16
    wq_rot = _rot_cols(wq, H).astype(bf)
    wk_rot = _rot_cols(wk, HKV).astype(bf)
    wq_b, wk_b, wv_b, wo_b = (w.astype(bf) for w in (wq, wk, wv, wo))
    ln1 = ln1_w.reshape(1, D)
    ln2 = ln2_w.reshape(1, D)
    wr_pad = jnp.pad(w_router, ((0, 0), (0, EPAD - E)))

    n1 = S // BS_QKV
    row_spec = lambda w: pl.BlockSpec((BS_QKV, w), lambda t: (t, 0))
    full_spec = lambda a, b: pl.BlockSpec((a, b), lambda t: (0, 0))
    q2d, k2d, v2d = pl.pallas_call(
        _qkv_kernel,
        grid=(n1,),
        in_specs=[
            row_spec(D), full_spec(1, D),
            full_spec(D, H * HD), full_spec(D, H * HD),
            full_spec(D, HKV * HD), full_spec(D, HKV * HD),
            full_spec(D, HKV * HD),
            row_spec(H * HD), row_spec(H * HD),
            row_spec(HKV * HD), row_spec(HKV * HD),
        ],
        out_specs=[row_spec(H * HD), row_spec(HKV * HD), row_spec(HKV * HD)],
        out_shape=[
            jax.ShapeDtypeStruct((S, H * HD), bf),
            jax.ShapeDtypeStruct((S, HKV * HD), bf),
            jax.ShapeDtypeStruct((S, HKV * HD), bf),
        ],
    )(x, ln1, wq_b, wq_rot, wk_b, wk_rot, wv_b, cosq, sinq, cosk, sink)

    q = q2d.reshape(S, H, HD).transpose(1, 0, 2)
    kh = k2d.reshape(S, HKV, HD).transpose(1, 0, 2)
    vh = v2d.reshape(S, HKV, HD).transpose(1, 0, 2)

    rep = H // HKV
    ctx = pl.pallas_call(
        _attn_kernel,
        grid=(H, S // BQ),
        in_specs=[
            pl.BlockSpec((1, BQ, HD), lambda h, i: (h, i, 0)),
            pl.BlockSpec((1, S, HD), lambda h, i: (h // rep, 0, 0)),
            pl.BlockSpec((1, S, HD), lambda h, i: (h // rep, 0, 0)),
        ],
        out_specs=pl.BlockSpec((1, BQ, HD), lambda h, i: (h, i, 0)),
        out_shape=jax.ShapeDtypeStruct((H, S, HD), bf),
    )(q, kh, vh)

    ctx2d = ctx.transpose(1, 0, 2).reshape(S, H * HD)

    n3 = S // BS_POST
    rs3 = lambda w: pl.BlockSpec((BS_POST, w), lambda t: (t, 0))
    hid, h2, pmat, oh0, oh1, cnt = pl.pallas_call(
        _post_kernel,
        grid=(n3,),
        in_specs=[
            rs3(D), rs3(D),
            pl.BlockSpec((D, D), lambda t: (0, 0)),
            pl.BlockSpec((1, D), lambda t: (0, 0)),
            pl.BlockSpec((D, EPAD), lambda t: (0, 0)),
        ],
        out_specs=[rs3(D), rs3(D), rs3(EPAD), rs3(EPAD), rs3(EPAD),
                   pl.BlockSpec((1, EPAD), lambda t: (0, 0))],
        out_shape=[
            jax.ShapeDtypeStruct((S, D), jnp.float32),
            jax.ShapeDtypeStruct((S, D), jnp.float32),
            jax.ShapeDtypeStruct((S, EPAD), jnp.float32),
            jax.ShapeDtypeStruct((S, EPAD), jnp.float32),
            jax.ShapeDtypeStruct((S, EPAD), jnp.float32),
            jax.ShapeDtypeStruct((1, EPAD), jnp.float32),
        ],
        scratch_shapes=[pltpu.VMEM((1, EPAD), jnp.float32)],
    )(ctx2d, x, wo_b, ln2, wr_pad)

    oh = jnp.concatenate([oh0, oh1], axis=0)
    cols128 = jnp.arange(EPAD, dtype=jnp.float32)
    lt = (cols128[:, None] < cols128[None, :]).astype(jnp.float32)
    rows_b = jnp.arange(BS_POS, dtype=jnp.float32)
    ltb = (rows_b[None, :] < rows_b[:, None]).astype(jnp.float32)

    pos2d, offs_f = pl.pallas_call(
        _pos_kernel,
        grid=(NTOK // BS_POS,),
        in_specs=[
            pl.BlockSpec((BS_POS, EPAD), lambda t: (t, 0)),
            pl.BlockSpec((1, EPAD), lambda t: (0, 0)),
            pl.BlockSpec((EPAD, EPAD), lambda t: (0, 0)),
            pl.BlockSpec((BS_POS, BS_POS), lambda t: (0, 0)),
        ],
        out_specs=[pl.BlockSpec((BS_POS, 1), lambda t: (t, 0)),
                   pl.BlockSpec((1, EPAD), lambda t: (0, 0))],
        out_shape=[jax.ShapeDtypeStruct((NTOK, 1), jnp.int32),
                   jax.ShapeDtypeStruct((1, EPAD), jnp.float32)],
        scratch_shapes=[pltpu.VMEM((1, EPAD), jnp.float32),
                        pltpu.VMEM((1, EPAD), jnp.float32)],
    )(oh, cnt, lt, ltb)

    pos = pos2d.reshape(NTOK)
    offs9 = jnp.concatenate(
        [offs_f[0, :E].astype(jnp.int32), jnp.array([NTOK], jnp.int32)])

    hs = _sc_permute(h2, pos, scatter=True)

    ys = pl.pallas_call(
        _moe_kernel,
        grid_spec=pltpu.PrefetchScalarGridSpec(
            num_scalar_prefetch=1,
            grid=(E, NF, NT),
            in_specs=[
                pl.BlockSpec((NTOK, D), lambda e, f, t, offs: (0, 0)),
                pl.BlockSpec((1, D, BF), lambda e, f, t, offs: (e, 0, f)),
                pl.BlockSpec((1, D, BF), lambda e, f, t, offs: (e, 0, f)),
                pl.BlockSpec((1, BF, D), lambda e, f, t, offs: (e, f, 0)),
            ],
            out_specs=pl.BlockSpec((NTOK, D), lambda e, f, t, offs: (0, 0)),
            scratch_shapes=[pltpu.VMEM((D, BF), bf),
                            pltpu.VMEM((D, BF), bf),
                            pltpu.VMEM((BF, D), bf)],
        ),
        out_shape=jax.ShapeDtypeStruct((NTOK, D), bf),
    )(offs9, hs, wg, wu, wd)

    g2 = _sc_permute(ys, pos, scatter=False)

    g2k = g2.reshape(K, S, D)
    out2d = pl.pallas_call(
        _combine_kernel,
        grid=(n3,),
        in_specs=[
            rs3(D), rs3(EPAD),
            pl.BlockSpec((1, BS_POST, D), lambda t: (0, t, 0)),
            pl.BlockSpec((1, BS_POST, D), lambda t: (1, t, 0)),
        ],
        out_specs=rs3(D),
        out_shape=jax.ShapeDtypeStruct((S, D), jnp.float32),
    )(hid, pmat, g2k, g2k)

    return out2d.reshape(B, S, D)


# speedup vs baseline: 1.2740x; 1.2740x over previous
import functools

import jax
import jax.numpy as jnp
from jax import lax
from jax.experimental import pallas as pl
from jax.experimental.pallas import tpu as pltpu
from jax.experimental.pallas import tpu_sc as plsc

B, S, D = 1, 2048, 768
H, HKV, HD = 12, 4, 64
E, K, FF = 8, 2, 2048
EPS = 1e-5
THETA = 1e6

BS_QKV = 256
BQ = 512
BK = 1024
BS_POST = 256
BS_POS = 512
EPAD = 128

NTOK = S * K
TM = 256
NT = NTOK // TM
BF = 1024
NF = FF // BF

NSC_WORKERS = 32
CHUNK = NTOK // NSC_WORKERS

NEG = -1e30


def _dot(a, b):
    return jnp.dot(a, b, preferred_element_type=jnp.float32)


def _qkv_kernel(x_ref, ln1_ref, wq_ref, wqr_ref, wk_ref, wkr_ref, wv_ref,
                cosq_ref, sinq_ref, cosk_ref, sink_ref,
                q_ref, k_ref, v_ref):
    x = x_ref[...]
    var = jnp.mean(x * x, axis=1, keepdims=True)
    xn = (x * jax.lax.rsqrt(var + EPS) * ln1_ref[...]).astype(jnp.bfloat16)
    q = _dot(xn, wq_ref[...])
    qr = _dot(xn, wqr_ref[...])
    q_ref[...] = (q * cosq_ref[...] + qr * sinq_ref[...]).astype(jnp.bfloat16)
    k = _dot(xn, wk_ref[...])
    kr = _dot(xn, wkr_ref[...])
    k_ref[...] = (k * cosk_ref[...] + kr * sink_ref[...]).astype(jnp.bfloat16)
    v_ref[...] = _dot(xn, wv_ref[...]).astype(jnp.bfloat16)


def _attn_kernel(q_ref, k_ref, v_ref, o_ref):
    i = pl.program_id(1)
    q = q_ref[0]
    nj = (i * BQ + BQ - 1) // BK + 1

    def body(j, carry):
        m_old, l_old, acc_old = carry
        base = pl.multiple_of(j * BK, BK)
        k = k_ref[0, pl.ds(base, BK), :]
        s = _dot(q, k.T) * (1.0 / 8.0)
        rows = i * BQ + jax.lax.broadcasted_iota(jnp.int32, (BQ, BK), 0)
        cols = j * BK + jax.lax.broadcasted_iota(jnp.int32, (BQ, BK), 1)
        s = jnp.where(cols <= rows, s, NEG)
        m_new = jnp.maximum(m_old, jnp.max(s, axis=1, keepdims=True))
        alpha = jnp.exp(m_old - m_new)
        p = jnp.exp(s - m_new)
        l_new = l_old * alpha + jnp.sum(p, axis=1, keepdims=True)
        v = v_ref[0, pl.ds(base, BK), :]
        acc_new = acc_old * alpha + _dot(p.astype(jnp.bfloat16), v)
        return m_new, l_new, acc_new

    m0 = jnp.full((BQ, 1), NEG, jnp.float32)
    l0 = jnp.zeros((BQ, 1), jnp.float32)
    a0 = jnp.zeros((BQ, HD), jnp.float32)
    m, l, acc = jax.lax.fori_loop(0, nj, body, (m0, l0, a0))
    o_ref[0] = (acc / l).astype(jnp.bfloat16)


def _post_kernel(ctx_ref, res_ref, wo_ref, ln2_ref, wr_ref,
                 hid_ref, h2_ref, p_ref, oh0_ref, oh1_ref, cnt_ref,
                 cnt_acc):
    t = pl.program_id(0)
    hid = res_ref[...] + _dot(ctx_ref[...], wo_ref[...])
    hid_ref[...] = hid
    var = jnp.mean(hid * hid, axis=1, keepdims=True)
    h2 = hid * jax.lax.rsqrt(var + EPS) * ln2_ref[...]
    h2_ref[...] = h2
    logits = _dot(h2, wr_ref[...])
    cols = jax.lax.broadcasted_iota(jnp.int32, (BS_POST, EPAD), 1)
    logits = jnp.where(cols < E, logits, NEG)
    m1 = jnp.max(logits, axis=1, keepdims=True)
    i1 = jnp.min(jnp.where(logits == m1, cols, EPAD), axis=1, keepdims=True)
    masked = jnp.where(cols == i1, NEG, logits)
    m2 = jnp.max(masked, axis=1, keepdims=True)
    i2 = jnp.min(jnp.where(masked == m2, cols, EPAD), axis=1, keepdims=True)
    p1 = 1.0 / (1.0 + jnp.exp(m2 - m1))
    p2 = 1.0 - p1
    p_ref[...] = jnp.where(cols == 0, p1, 0.0) + jnp.where(cols == 1, p2, 0.0)
    oh0 = jnp.where(cols == i1, 1.0, 0.0)
    oh1 = jnp.where(cols == i2, 1.0, 0.0)
    oh0_ref[...] = oh0
    oh1_ref[...] = oh1

    @pl.when(t == 0)
    def _():
        cnt_acc[...] = jnp.zeros_like(cnt_acc)

    cnt_acc[...] += jnp.sum(oh0 + oh1, axis=0, keepdims=True)

    @pl.when(t == pl.num_programs(0) - 1)
    def _():
        cnt_ref[...] = cnt_acc[...]


def _pos_kernel(oh_ref, cnt_ref, lt_ref, ltb_ref, pos_ref, offs_ref,
                offs_acc, carry_acc):
    t = pl.program_id(0)

    @pl.when(t == 0)
    def _():
        offs_acc[...] = jnp.dot(cnt_ref[...], lt_ref[...],
                                preferred_element_type=jnp.float32,
                                precision=jax.lax.Precision.HIGHEST)
        carry_acc[...] = jnp.zeros_like(carry_acc)
        offs_ref[...] = offs_acc[...]

    oh = oh_ref[...]
    ecs = _dot(ltb_ref[...], oh)
    posmat = oh * (offs_acc[...] + carry_acc[...] + ecs)
    pos = jnp.sum(posmat, axis=1, keepdims=True)
    pos_ref[...] = pos.astype(jnp.int32)
    carry_acc[...] += jnp.sum(oh, axis=0, keepdims=True)


def _sc_scatter_body(h2_hbm, pos_hbm, hs_hbm, idx_v, rows_v, sem):
    wid = lax.axis_index("s") * 2 + lax.axis_index("c")
    base = wid * CHUNK
    tbase = base % S
    pltpu.sync_copy(pos_hbm.at[pl.ds(base, CHUNK)], idx_v)
    pltpu.sync_copy(h2_hbm.at[pl.ds(tbase, CHUNK)], rows_v)
    pltpu.async_copy(rows_v, hs_hbm.at[idx_v], sem).wait()


def _sc_gather_body(ys_hbm, pos_hbm, g2_hbm, idx_v, rows_v, sem):
    wid = lax.axis_index("s") * 2 + lax.axis_index("c")
    base = wid * CHUNK
    pltpu.sync_copy(pos_hbm.at[pl.ds(base, CHUNK)], idx_v)
    pltpu.async_copy(ys_hbm.at[idx_v], rows_v, sem).wait()
    pltpu.sync_copy(rows_v, g2_hbm.at[pl.ds(base, CHUNK)])


def _moe_kernel(offs_ref, hs_ref, wg_ref, wu_ref, wd_ref, ys_ref,
                wgs, wus, wds):
    e = pl.program_id(0)
    f = pl.program_id(1)
    t = pl.program_id(2)

    @pl.when((e == 0) & (f == 0) & (t == 0))
    def _():
        ys_ref[...] = jnp.zeros_like(ys_ref)

    start = offs_ref[e]
    end = offs_ref[e + 1]
    lo = jnp.maximum(t * TM, start)
    hi = jnp.minimum((t + 1) * TM, end)

    @pl.when((hi > lo) & (t == start // TM))
    def _():
        wgs[...] = wg_ref[0].astype(jnp.bfloat16)
        wus[...] = wu_ref[0].astype(jnp.bfloat16)
        wds[...] = wd_ref[0].astype(jnp.bfloat16)

    @pl.when(hi > lo)
    def _():
        rows = hs_ref[pl.ds(t * TM, TM), :].astype(jnp.bfloat16)
        g = jax.nn.silu(_dot(rows, wgs[...])) * _dot(rows, wus[...])
        y = _dot(g.astype(jnp.bfloat16), wds[...])
        rid = t * TM + jax.lax.broadcasted_iota(jnp.int32, (TM, 1), 0)
        mask = (rid >= lo) & (rid < hi)
        ys_ref[pl.ds(t * TM, TM), :] += jnp.where(mask, y, 0.0)


def _combine_kernel(hid_ref, p_ref, g0_ref, g1_ref, out_ref):
    p1 = p_ref[:, 0:1]
    p2 = p_ref[:, 1:2]
    out_ref[...] = hid_ref[...] + p1 * g0_ref[0] + p2 * g1_ref[0]


def _sc_permute(src, pos, scatter):
    mesh = plsc.VectorSubcoreMesh(core_axis_name="c", subcore_axis_name="s",
                                  num_cores=2, num_subcores=16)
    body = _sc_scatter_body if scatter else _sc_gather_body
    return pl.kernel(
        body, mesh=mesh,
        out_type=jax.ShapeDtypeStruct((NTOK, D), jnp.float32),
        scratch_types=[
            pltpu.VMEM((CHUNK,), jnp.int32),
            pltpu.VMEM((CHUNK, D), jnp.float32),
            pltpu.SemaphoreType.DMA,
        ],
    )(src, pos)


def _rot_cols(w, heads):
    w4 = w.reshape(w.shape[0], heads, 2, HD // 2)
    return jnp.concatenate([-w4[:, :, 1], w4[:, :, 0]], axis=2).reshape(w.shape)


def kernel(hidden_states, position_ids, ln1_w, ln2_w, wq, wk, wv, wo,
           w_router, wg, wu, wd):
    x = hidden_states.reshape(S, D)

    inv_freq = 1.0 / (THETA ** (jnp.arange(0, HD, 2, dtype=jnp.float32) / HD))
    pos_f = position_ids.reshape(S).astype(jnp.float32)
    freqs = pos_f[:, None] * inv_freq[None, :]
    cos1 = jnp.cos(freqs)
    sin1 = jnp.sin(freqs)
    cosh = jnp.concatenate([cos1, cos1], axis=1)
    sinh = jnp.concatenate([sin1, sin1], axis=1)
    cosq = jnp.tile(cosh, (1, H))
    sinq = jnp.tile(sinh, (1, H))
    cosk = jnp.tile(cosh, (1, HKV))
    sink = jnp.tile(sinh, (1, HKV))

    bf = jnp.bfloat16
    wq_rot = _rot_cols(wq, H).astype(bf)
    wk_rot = _rot_cols(wk, HKV).astype(bf)
    wq_b, wk_b, wv_b, wo_b = (w.astype(bf) for w in (wq, wk, wv, wo))
    ln1 = ln1_w.reshape(1, D)
    ln2 = ln2_w.reshape(1, D)
    wr_pad = jnp.pad(w_router, ((0, 0), (0, EPAD - E)))

    n1 = S // BS_QKV
    row_spec = lambda w: pl.BlockSpec((BS_QKV, w), lambda t: (t, 0))
    full_spec = lambda a, b: pl.BlockSpec((a, b), lambda t: (0, 0))
    q2d, k2d, v2d = pl.pallas_call(
        _qkv_kernel,
        grid=(n1,),
        in_specs=[
            row_spec(D), full_spec(1, D),
            full_spec(D, H * HD), full_spec(D, H * HD),
            full_spec(D, HKV * HD), full_spec(D, HKV * HD),
            full_spec(D, HKV * HD),
            row_spec(H * HD), row_spec(H * HD),
            row_spec(HKV * HD), row_spec(HKV * HD),
        ],
        out_specs=[row_spec(H * HD), row_spec(HKV * HD), row_spec(HKV * HD)],
        out_shape=[
            jax.ShapeDtypeStruct((S, H * HD), bf),
            jax.ShapeDtypeStruct((S, HKV * HD), bf),
            jax.ShapeDtypeStruct((S, HKV * HD), bf),
        ],
    )(x, ln1, wq_b, wq_rot, wk_b, wk_rot, wv_b, cosq, sinq, cosk, sink)

    q = q2d.reshape(S, H, HD).transpose(1, 0, 2)
    kh = k2d.reshape(S, HKV, HD).transpose(1, 0, 2)
    vh = v2d.reshape(S, HKV, HD).transpose(1, 0, 2)

    rep = H // HKV
    ctx = pl.pallas_call(
        _attn_kernel,
        grid=(H, S // BQ),
        in_specs=[
            pl.BlockSpec((1, BQ, HD), lambda h, i: (h, i, 0)),
            pl.BlockSpec((1, S, HD), lambda h, i: (h // rep, 0, 0)),
            pl.BlockSpec((1, S, HD), lambda h, i: (h // rep, 0, 0)),
        ],
        out_specs=pl.BlockSpec((1, BQ, HD), lambda h, i: (h, i, 0)),
        out_shape=jax.ShapeDtypeStruct((H, S, HD), bf),
    )(q, kh, vh)

    ctx2d = ctx.transpose(1, 0, 2).reshape(S, H * HD)

    n3 = S // BS_POST
    rs3 = lambda w: pl.BlockSpec((BS_POST, w), lambda t: (t, 0))
    hid, h2, pmat, oh0, oh1, cnt = pl.pallas_call(
        _post_kernel,
        grid=(n3,),
        in_specs=[
            rs3(D), rs3(D),
            pl.BlockSpec((D, D), lambda t: (0, 0)),
            pl.BlockSpec((1, D), lambda t: (0, 0)),
            pl.BlockSpec((D, EPAD), lambda t: (0, 0)),
        ],
        out_specs=[rs3(D), rs3(D), rs3(EPAD), rs3(EPAD), rs3(EPAD),
                   pl.BlockSpec((1, EPAD), lambda t: (0, 0))],
        out_shape=[
            jax.ShapeDtypeStruct((S, D), jnp.float32),
            jax.ShapeDtypeStruct((S, D), jnp.float32),
            jax.ShapeDtypeStruct((S, EPAD), jnp.float32),
            jax.ShapeDtypeStruct((S, EPAD), jnp.float32),
            jax.ShapeDtypeStruct((S, EPAD), jnp.float32),
            jax.ShapeDtypeStruct((1, EPAD), jnp.float32),
        ],
        scratch_shapes=[pltpu.VMEM((1, EPAD), jnp.float32)],
    )(ctx2d, x, wo_b, ln2, wr_pad)

    oh = jnp.concatenate([oh0, oh1], axis=0)
    cols128 = jnp.arange(EPAD, dtype=jnp.float32)
    lt = (cols128[:, None] < cols128[None, :]).astype(jnp.float32)
    rows_b = jnp.arange(BS_POS, dtype=jnp.float32)
    ltb = (rows_b[None, :] < rows_b[:, None]).astype(jnp.float32)

    pos2d, offs_f = pl.pallas_call(
        _pos_kernel,
        grid=(NTOK // BS_POS,),
        in_specs=[
            pl.BlockSpec((BS_POS, EPAD), lambda t: (t, 0)),
            pl.BlockSpec((1, EPAD), lambda t: (0, 0)),
            pl.BlockSpec((EPAD, EPAD), lambda t: (0, 0)),
            pl.BlockSpec((BS_POS, BS_POS), lambda t: (0, 0)),
        ],
        out_specs=[pl.BlockSpec((BS_POS, 1), lambda t: (t, 0)),
                   pl.BlockSpec((1, EPAD), lambda t: (0, 0))],
        out_shape=[jax.ShapeDtypeStruct((NTOK, 1), jnp.int32),
                   jax.ShapeDtypeStruct((1, EPAD), jnp.float32)],
        scratch_shapes=[pltpu.VMEM((1, EPAD), jnp.float32),
                        pltpu.VMEM((1, EPAD), jnp.float32)],
    )(oh, cnt, lt, ltb)

    pos = pos2d.reshape(NTOK)
    offs9 = jnp.concatenate(
        [offs_f[0, :E].astype(jnp.int32), jnp.array([NTOK], jnp.int32)])

    hs = _sc_permute(h2, pos, scatter=True)

    ys = pl.pallas_call(
        _moe_kernel,
        grid_spec=pltpu.PrefetchScalarGridSpec(
            num_scalar_prefetch=1,
            grid=(E, NF, NT),
            in_specs=[
                pl.BlockSpec((NTOK, D), lambda e, f, t, offs: (0, 0)),
                pl.BlockSpec((1, D, BF), lambda e, f, t, offs: (e, 0, f)),
                pl.BlockSpec((1, D, BF), lambda e, f, t, offs: (e, 0, f)),
                pl.BlockSpec((1, BF, D), lambda e, f, t, offs: (e, f, 0)),
            ],
            out_specs=pl.BlockSpec((NTOK, D), lambda e, f, t, offs: (0, 0)),
            scratch_shapes=[pltpu.VMEM((D, BF), bf),
                            pltpu.VMEM((D, BF), bf),
                            pltpu.VMEM((BF, D), bf)],
        ),
        out_shape=jax.ShapeDtypeStruct((NTOK, D), jnp.float32),
    )(offs9, hs, wg, wu, wd)

    g2 = _sc_permute(ys, pos, scatter=False)

    g2k = g2.reshape(K, S, D)
    out2d = pl.pallas_call(
        _combine_kernel,
        grid=(n3,),
        in_specs=[
            rs3(D), rs3(EPAD),
            pl.BlockSpec((1, BS_POST, D), lambda t: (0, t, 0)),
            pl.BlockSpec((1, BS_POST, D), lambda t: (1, t, 0)),
        ],
        out_specs=rs3(D),
        out_shape=jax.ShapeDtypeStruct((S, D), jnp.float32),
    )(hid, pmat, g2k, g2k)

    return out2d.reshape(B, S, D)
